# baseline (device time: 49473 ns/iter reference)
import jax
import jax.numpy as jnp
from jax import lax
from jax.experimental import pallas as pl
from jax.experimental.pallas import tpu as pltpu

N_DEV = 4
SEG = 4


def kernel(x):
    _, m, n_tot = x.shape
    n_out = n_tot // N_DEV
    half = n_out // 2
    mseg = m // SEG

    def body(x_ref, out_ref, xv, comm_r, comm_l, contrib_r, contrib_l,
             load_sems, send_r, recv_r, send_l, recv_l):
        my = lax.axis_index("i")

        def run(k):
            left = (k - 1) % N_DEV
            right = (k + 1) % N_DEV
            diag = (k + 2) % N_DEV

            def load(idx, c, col0, ncol):
                return pltpu.make_async_copy(
                    x_ref.at[0, :, c * n_out + col0:c * n_out + col0 + ncol],
                    xv.at[c, :, col0:col0 + ncol],
                    load_sems.at[idx],
                )

            loads = [
                load(0, left, 0, half),
                load(1, right, half, half),
                load(2, diag, 0, n_out),
                load(3, right, 0, half),
                load(4, left, half, half),
                load(5, k, 0, n_out),
            ]
            for ld in loads:
                ld.start()

            barrier_sem = pltpu.get_barrier_semaphore()
            for nbr in (left, right):
                pl.semaphore_signal(
                    barrier_sem, inc=1,
                    device_id=(nbr,), device_id_type=pl.DeviceIdType.MESH,
                )
            pl.semaphore_wait(barrier_sem, 2)

            def rdma(direction, h, s):
                comm, ssem, rsem, dst_dev = (
                    (comm_r, send_r, recv_r, right) if direction == 0
                    else (comm_l, send_l, recv_l, left)
                )
                src_slot = 3 if h == 0 else h - 1
                rows = pl.ds(s * mseg, mseg)
                return pltpu.make_async_remote_copy(
                    src_ref=comm.at[src_slot, rows, :],
                    dst_ref=comm.at[h, rows, :],
                    send_sem=ssem.at[h, s],
                    recv_sem=rsem.at[h, s],
                    device_id=(dst_dev,),
                    device_id_type=pl.DeviceIdType.MESH,
                )

            loads[0].wait()
            loads[1].wait()
            for s in range(SEG):
                rows = pl.ds(s * mseg, mseg)
                comm_r[3, rows, :] = xv[left, rows, :half].astype(jnp.bfloat16)
                rdma(0, 0, s).start()
                comm_l[3, rows, :] = xv[right, rows, half:].astype(jnp.bfloat16)
                rdma(1, 0, s).start()

            loads[2].wait()
            contrib_r[0, :, :] = xv[diag, :, :half].astype(jnp.bfloat16)
            contrib_l[0, :, :] = xv[diag, :, half:].astype(jnp.bfloat16)
            loads[3].wait()
            contrib_r[1, :, :] = xv[right, :, :half].astype(jnp.bfloat16)
            loads[4].wait()
            contrib_l[1, :, :] = xv[left, :, half:].astype(jnp.bfloat16)

            for h in range(N_DEV - 2):
                for s in range(SEG):
                    rows = pl.ds(s * mseg, mseg)
                    rdma(0, h, s).wait_recv()
                    comm_r[h, rows, :] = (
                        comm_r[h, rows, :] + contrib_r[h, rows, :]
                    )
                    rdma(0, h + 1, s).start()
                    rdma(1, h, s).wait_recv()
                    comm_l[h, rows, :] = (
                        comm_l[h, rows, :] + contrib_l[h, rows, :]
                    )
                    rdma(1, h + 1, s).start()

            loads[5].wait()
            hl = N_DEV - 2
            for s in range(SEG):
                rows = pl.ds(s * mseg, mseg)
                rdma(0, hl, s).wait_recv()
                out_ref[rows, :half] = (
                    comm_r[hl, rows, :].astype(jnp.float32)
                    + xv[k, rows, :half]
                ).astype(jnp.bfloat16)
                rdma(1, hl, s).wait_recv()
                out_ref[rows, half:] = (
                    comm_l[hl, rows, :].astype(jnp.float32)
                    + xv[k, rows, half:]
                ).astype(jnp.bfloat16)

            for h in range(N_DEV - 1):
                for s in range(SEG):
                    rdma(0, h, s).wait_send()
                    rdma(1, h, s).wait_send()

        for k in range(N_DEV):
            pl.when(my == k)(lambda k=k: run(k))

    return pl.pallas_call(
        body,
        out_shape=jax.ShapeDtypeStruct((m, n_out), jnp.bfloat16),
        in_specs=[pl.BlockSpec(memory_space=pltpu.MemorySpace.HBM)],
        out_specs=pl.BlockSpec(memory_space=pltpu.VMEM),
        scratch_shapes=[
            pltpu.VMEM((4, m, n_out), jnp.float32),
            pltpu.VMEM((4, m, half), jnp.bfloat16),
            pltpu.VMEM((4, m, half), jnp.bfloat16),
            pltpu.VMEM((2, m, half), jnp.bfloat16),
            pltpu.VMEM((2, m, half), jnp.bfloat16),
            pltpu.SemaphoreType.DMA((6,)),
            pltpu.SemaphoreType.DMA((3, SEG)),
            pltpu.SemaphoreType.DMA((3, SEG)),
            pltpu.SemaphoreType.DMA((3, SEG)),
            pltpu.SemaphoreType.DMA((3, SEG)),
        ],
        compiler_params=pltpu.CompilerParams(collective_id=0),
    )(x)


# device time: 47342 ns/iter; 1.0450x vs baseline; 1.0450x over previous
import jax
import jax.numpy as jnp
from jax import lax
from jax.experimental import pallas as pl
from jax.experimental.pallas import tpu as pltpu

N_DEV = 4
SEG = 4


def kernel(x):
    _, m, n_tot = x.shape
    n_out = n_tot // N_DEV
    half = n_out // 2
    mseg = m // SEG

    def body(x_ref, out_ref, xv, comm_r, comm_l, contrib_r, contrib_l,
             load_sems, send_r, recv_r, send_l, recv_l):
        my = lax.axis_index("i")

        def run(k):
            left = (k - 1) % N_DEV
            right = (k + 1) % N_DEV
            diag = (k + 2) % N_DEV

            def load(s):
                rows = pl.ds(s * mseg, mseg)
                return pltpu.make_async_copy(
                    x_ref.at[0, rows, :], xv.at[rows, :], load_sems.at[s],
                )

            loads = [load(s) for s in range(SEG)]
            for ld in loads:
                ld.start()

            barrier_sem = pltpu.get_barrier_semaphore()
            for nbr in (left, right):
                pl.semaphore_signal(
                    barrier_sem, inc=1,
                    device_id=(nbr,), device_id_type=pl.DeviceIdType.MESH,
                )
            pl.semaphore_wait(barrier_sem, 2)

            def rdma(direction, h, s):
                comm, ssem, rsem, dst_dev = (
                    (comm_r, send_r, recv_r, right) if direction == 0
                    else (comm_l, send_l, recv_l, left)
                )
                src_slot = 3 if h == 0 else h - 1
                rows = pl.ds(s * mseg, mseg)
                return pltpu.make_async_remote_copy(
                    src_ref=comm.at[src_slot, rows, :],
                    dst_ref=comm.at[h, rows, :],
                    send_sem=ssem.at[h, s],
                    recv_sem=rsem.at[h, s],
                    device_id=(dst_dev,),
                    device_id_type=pl.DeviceIdType.MESH,
                )

            def cols(c, hi):
                lo = c * n_out + hi * half
                return slice(lo, lo + half)

            for s in range(SEG):
                rows = pl.ds(s * mseg, mseg)
                loads[s].wait()
                comm_r[3, rows, :] = xv[rows, cols(left, 0)].astype(
                    jnp.bfloat16)
                rdma(0, 0, s).start()
                comm_l[3, rows, :] = xv[rows, cols(right, 1)].astype(
                    jnp.bfloat16)
                rdma(1, 0, s).start()

            contrib_r[0, :, :] = xv[:, cols(diag, 0)].astype(jnp.bfloat16)
            contrib_l[0, :, :] = xv[:, cols(diag, 1)].astype(jnp.bfloat16)
            contrib_r[1, :, :] = xv[:, cols(right, 0)].astype(jnp.bfloat16)
            contrib_l[1, :, :] = xv[:, cols(left, 1)].astype(jnp.bfloat16)

            for h in range(N_DEV - 2):
                for s in range(SEG):
                    rows = pl.ds(s * mseg, mseg)
                    rdma(0, h, s).wait_recv()
                    comm_r[h, rows, :] = (
                        comm_r[h, rows, :] + contrib_r[h, rows, :]
                    )
                    rdma(0, h + 1, s).start()
                    rdma(1, h, s).wait_recv()
                    comm_l[h, rows, :] = (
                        comm_l[h, rows, :] + contrib_l[h, rows, :]
                    )
                    rdma(1, h + 1, s).start()

            hl = N_DEV - 2
            for s in range(SEG):
                rows = pl.ds(s * mseg, mseg)
                rdma(0, hl, s).wait_recv()
                out_ref[rows, :half] = (
                    comm_r[hl, rows, :].astype(jnp.float32)
                    + xv[rows, cols(k, 0)]
                ).astype(jnp.bfloat16)
                rdma(1, hl, s).wait_recv()
                out_ref[rows, half:] = (
                    comm_l[hl, rows, :].astype(jnp.float32)
                    + xv[rows, cols(k, 1)]
                ).astype(jnp.bfloat16)

            for h in range(N_DEV - 1):
                for s in range(SEG):
                    rdma(0, h, s).wait_send()
                    rdma(1, h, s).wait_send()

        for k in range(N_DEV):
            pl.when(my == k)(lambda k=k: run(k))

    return pl.pallas_call(
        body,
        out_shape=jax.ShapeDtypeStruct((m, n_out), jnp.bfloat16),
        in_specs=[pl.BlockSpec(memory_space=pltpu.MemorySpace.HBM)],
        out_specs=pl.BlockSpec(memory_space=pltpu.VMEM),
        scratch_shapes=[
            pltpu.VMEM((m, n_tot), jnp.float32),
            pltpu.VMEM((4, m, half), jnp.bfloat16),
            pltpu.VMEM((4, m, half), jnp.bfloat16),
            pltpu.VMEM((2, m, half), jnp.bfloat16),
            pltpu.VMEM((2, m, half), jnp.bfloat16),
            pltpu.SemaphoreType.DMA((SEG,)),
            pltpu.SemaphoreType.DMA((3, SEG)),
            pltpu.SemaphoreType.DMA((3, SEG)),
            pltpu.SemaphoreType.DMA((3, SEG)),
            pltpu.SemaphoreType.DMA((3, SEG)),
        ],
        compiler_params=pltpu.CompilerParams(collective_id=0),
    )(x)


# device time: 41737 ns/iter; 1.1854x vs baseline; 1.1343x over previous
import jax
import jax.numpy as jnp
from jax import lax
from jax.experimental import pallas as pl
from jax.experimental.pallas import tpu as pltpu

N_DEV = 4
SEG = 4


def kernel(x):
    _, m, n_tot = x.shape
    n_out = n_tot // N_DEV
    half = n_out // 2
    mseg = m // SEG

    def body(x_ref, out_ref, xv, out_stage, comm_r, comm_l, contrib_r,
             contrib_l, load_sems, store_sems, send_r, recv_r,
             send_l, recv_l):
        my = lax.axis_index("i")

        def run(k):
            left = (k - 1) % N_DEV
            right = (k + 1) % N_DEV
            diag = (k + 2) % N_DEV

            def load(s):
                rows = pl.ds(s * mseg, mseg)
                return pltpu.make_async_copy(
                    x_ref.at[0, rows, :], xv.at[rows, :], load_sems.at[s],
                )

            loads = [load(s) for s in range(SEG)]
            for ld in loads:
                ld.start()

            barrier_sem = pltpu.get_barrier_semaphore()
            for nbr in (left, right):
                pl.semaphore_signal(
                    barrier_sem, inc=1,
                    device_id=(nbr,), device_id_type=pl.DeviceIdType.MESH,
                )
            pl.semaphore_wait(barrier_sem, 2)

            def rdma(direction, h, s):
                comm, ssem, rsem, dst_dev = (
                    (comm_r, send_r, recv_r, right) if direction == 0
                    else (comm_l, send_l, recv_l, left)
                )
                src_slot = 3 if h == 0 else h - 1
                rows = pl.ds(s * mseg, mseg)
                return pltpu.make_async_remote_copy(
                    src_ref=comm.at[src_slot, rows, :],
                    dst_ref=comm.at[h, rows, :],
                    send_sem=ssem.at[h, s],
                    recv_sem=rsem.at[h, s],
                    device_id=(dst_dev,),
                    device_id_type=pl.DeviceIdType.MESH,
                )

            def cols(c, hi):
                lo = c * n_out + hi * half
                return slice(lo, lo + half)

            for s in range(SEG):
                rows = pl.ds(s * mseg, mseg)
                loads[s].wait()
                comm_r[3, rows, :] = xv[rows, cols(left, 0)].astype(
                    jnp.bfloat16)
                rdma(0, 0, s).start()
                comm_l[3, rows, :] = xv[rows, cols(right, 1)].astype(
                    jnp.bfloat16)
                rdma(1, 0, s).start()
                contrib_r[0, rows, :] = xv[rows, cols(diag, 0)].astype(
                    jnp.bfloat16)
                contrib_l[0, rows, :] = xv[rows, cols(diag, 1)].astype(
                    jnp.bfloat16)
                contrib_r[1, rows, :] = xv[rows, cols(right, 0)].astype(
                    jnp.bfloat16)
                contrib_l[1, rows, :] = xv[rows, cols(left, 1)].astype(
                    jnp.bfloat16)

            for h in range(N_DEV - 2):
                for s in range(SEG):
                    rows = pl.ds(s * mseg, mseg)
                    rdma(0, h, s).wait_recv()
                    comm_r[h, rows, :] = (
                        comm_r[h, rows, :] + contrib_r[h, rows, :]
                    )
                    rdma(0, h + 1, s).start()
                    rdma(1, h, s).wait_recv()
                    comm_l[h, rows, :] = (
                        comm_l[h, rows, :] + contrib_l[h, rows, :]
                    )
                    rdma(1, h + 1, s).start()

            hl = N_DEV - 2
            stores = []
            for s in range(SEG):
                rows = pl.ds(s * mseg, mseg)
                rdma(0, hl, s).wait_recv()
                out_stage[rows, :half] = (
                    comm_r[hl, rows, :].astype(jnp.float32)
                    + xv[rows, cols(k, 0)]
                ).astype(jnp.bfloat16)
                rdma(1, hl, s).wait_recv()
                out_stage[rows, half:] = (
                    comm_l[hl, rows, :].astype(jnp.float32)
                    + xv[rows, cols(k, 1)]
                ).astype(jnp.bfloat16)
                st = pltpu.make_async_copy(
                    out_stage.at[rows, :], out_ref.at[rows, :],
                    store_sems.at[s],
                )
                st.start()
                stores.append(st)

            for st in stores:
                st.wait()
            for h in range(N_DEV - 1):
                for s in range(SEG):
                    rdma(0, h, s).wait_send()
                    rdma(1, h, s).wait_send()

        for k in range(N_DEV):
            pl.when(my == k)(lambda k=k: run(k))

    return pl.pallas_call(
        body,
        out_shape=jax.ShapeDtypeStruct((m, n_out), jnp.bfloat16),
        in_specs=[pl.BlockSpec(memory_space=pltpu.MemorySpace.HBM)],
        out_specs=pl.BlockSpec(memory_space=pltpu.MemorySpace.HBM),
        scratch_shapes=[
            pltpu.VMEM((m, n_tot), jnp.float32),
            pltpu.VMEM((m, n_out), jnp.bfloat16),
            pltpu.VMEM((4, m, half), jnp.bfloat16),
            pltpu.VMEM((4, m, half), jnp.bfloat16),
            pltpu.VMEM((2, m, half), jnp.bfloat16),
            pltpu.VMEM((2, m, half), jnp.bfloat16),
            pltpu.SemaphoreType.DMA((SEG,)),
            pltpu.SemaphoreType.DMA((SEG,)),
            pltpu.SemaphoreType.DMA((3, SEG)),
            pltpu.SemaphoreType.DMA((3, SEG)),
            pltpu.SemaphoreType.DMA((3, SEG)),
            pltpu.SemaphoreType.DMA((3, SEG)),
        ],
        compiler_params=pltpu.CompilerParams(collective_id=0),
    )(pltpu.with_memory_space_constraint(x, pltpu.MemorySpace.HBM))
